# baseline (speedup 1.0000x reference)
import functools

import jax
import jax.numpy as jnp
from jax import lax
from jax.experimental import pallas as pl
from jax.experimental.pallas import tpu as pltpu
from jax.experimental.pallas import tpu_sc as plsc

_VOCAB = 100000
_EMB = 16
_CTX = 10
_HID = 256
_B = 1024


_NC = 2
_NS = 16
_NW = _NC * _NS
_N_IDX = _B * _CTX
_PER_W = _N_IDX // _NW
_L = 16
_WIDE = 128
_ROWS_PER_WIDE = _WIDE // _EMB
_VW = _VOCAB // _ROWS_PER_WIDE
_OUT_WPW = _PER_W * _EMB // _WIDE
_CHUNK = 80
_NCHUNK = _PER_W // _CHUNK
_GROUPS = _PER_W // _L
_GPR = _CHUNK // _L


@functools.partial(
    pl.kernel,
    mesh=plsc.VectorSubcoreMesh(core_axis_name="c", subcore_axis_name="s"),
    out_type=jax.ShapeDtypeStruct((_N_IDX * _EMB // _WIDE, _WIDE), jnp.float32),
    scratch_types=[
        pltpu.VMEM((_NCHUNK, _CHUNK), jnp.int32),
        pltpu.VMEM((_NCHUNK, _CHUNK), jnp.int32),
        pltpu.VMEM((_PER_W, _WIDE), jnp.float32),
        pltpu.VMEM((_OUT_WPW, _WIDE), jnp.float32),
        pltpu.SemaphoreType.DMA,
    ],
)
def _sc_gather(table_hbm, idx_hbm, out_hbm, idx_v, k_v, rows_v, out_v, sem):
    wid = lax.axis_index("s") * _NC + lax.axis_index("c")
    pltpu.sync_copy(idx_hbm.at[wid], idx_v)
    for g in range(_GROUPS):
        r16 = idx_v[g // _GPR, pl.ds((g % _GPR) * _L, _L)]
        k_v[g // _GPR, pl.ds((g % _GPR) * _L, _L)] = r16 >> 3
    copies = [
        pltpu.async_copy(
            table_hbm.at[k_v.at[j]],
            rows_v.at[pl.ds(j * _CHUNK, _CHUNK)],
            sem,
        )
        for j in range(_NCHUNK)
    ]
    for c in copies:
        c.wait()
    for g in range(_GROUPS):
        r16 = idx_v[g // _GPR, pl.ds((g % _GPR) * _L, _L)]
        for k in range(_L):
            i = g * _L + k
            seg = (r16[k] & 7) * _EMB
            out_v[i // _ROWS_PER_WIDE, pl.ds((i % _ROWS_PER_WIDE) * _EMB, _EMB)] = (
                rows_v[i, pl.ds(seg, _EMB)]
            )
    pltpu.sync_copy(out_v, out_hbm.at[pl.ds(wid * _OUT_WPW, _OUT_WPW)])


_NVT = 1024
_T = (_VOCAB + _NVT - 1) // _NVT


def _hidden(emb_ref, w1_ref, b1_ref):
    h = jnp.dot(
        emb_ref[...].astype(jnp.bfloat16),
        w1_ref[...].astype(jnp.bfloat16),
        preferred_element_type=jnp.float32,
    )
    return jnp.maximum(h + b1_ref[...], 0.0).astype(jnp.bfloat16)


def _lse_body(emb_ref, w1_ref, b1_ref, w2_ref, b2_ref, lse_ref, h_ref, m_ref, s_ref):
    t = pl.program_id(0)

    @pl.when(t == 0)
    def _():
        h_ref[...] = _hidden(emb_ref, w1_ref, b1_ref)
        m_ref[...] = jnp.full(m_ref.shape, -jnp.inf, jnp.float32)
        s_ref[...] = jnp.zeros(s_ref.shape, jnp.float32)

    logits = jnp.dot(
        h_ref[...], w2_ref[...].astype(jnp.bfloat16),
        preferred_element_type=jnp.float32,
    ) + b2_ref[...]
    col = t * _NVT + lax.broadcasted_iota(jnp.int32, (1, _NVT), 1)
    logits = jnp.where(col < _VOCAB, logits, -jnp.inf)
    m_old = m_ref[...]
    m_new = jnp.maximum(m_old, jnp.max(logits, axis=1, keepdims=True))
    s_new = s_ref[...] * jnp.exp(m_old - m_new) + jnp.sum(
        jnp.exp(logits - m_new), axis=1, keepdims=True
    )
    m_ref[...] = m_new
    s_ref[...] = s_new

    @pl.when(t == _T - 1)
    def _():
        lse_ref[...] = m_new + jnp.log(s_new)


_lse_call = pl.pallas_call(
    _lse_body,
    grid=(_T,),
    in_specs=[
        pl.BlockSpec((_B, _CTX * _EMB), lambda t: (0, 0)),
        pl.BlockSpec((_CTX * _EMB, _HID), lambda t: (0, 0)),
        pl.BlockSpec((1, _HID), lambda t: (0, 0)),
        pl.BlockSpec((_HID, _NVT), lambda t: (0, t)),
        pl.BlockSpec((1, _NVT), lambda t: (0, t)),
    ],
    out_specs=pl.BlockSpec((_B, 1), lambda t: (0, 0)),
    out_shape=jax.ShapeDtypeStruct((_B, 1), jnp.float32),
    scratch_shapes=[
        pltpu.VMEM((_B, _HID), jnp.bfloat16),
        pltpu.VMEM((_B, 1), jnp.float32),
        pltpu.VMEM((_B, 1), jnp.float32),
    ],
)


def _out_body(emb_ref, w1_ref, b1_ref, w2_ref, b2_ref, lse_ref, out_ref, h_ref):
    t = pl.program_id(0)

    @pl.when(t == 0)
    def _():
        h_ref[...] = _hidden(emb_ref, w1_ref, b1_ref)

    logits = jnp.dot(
        h_ref[...], w2_ref[...].astype(jnp.bfloat16),
        preferred_element_type=jnp.float32,
    ) + b2_ref[...]
    out_ref[...] = logits - lse_ref[...]


_out_call = pl.pallas_call(
    _out_body,
    grid=(_T,),
    in_specs=[
        pl.BlockSpec((_B, _CTX * _EMB), lambda t: (0, 0)),
        pl.BlockSpec((_CTX * _EMB, _HID), lambda t: (0, 0)),
        pl.BlockSpec((1, _HID), lambda t: (0, 0)),
        pl.BlockSpec((_HID, _NVT), lambda t: (0, t)),
        pl.BlockSpec((1, _NVT), lambda t: (0, t)),
        pl.BlockSpec((_B, 1), lambda t: (0, 0)),
    ],
    out_specs=pl.BlockSpec((_B, _NVT), lambda t: (0, t)),
    out_shape=jax.ShapeDtypeStruct((_B, _VOCAB), jnp.float32),
    scratch_shapes=[
        pltpu.VMEM((_B, _HID), jnp.bfloat16),
    ],
)


def kernel(inputs, emb, W1, b1, W2, b2):
    idx = inputs.astype(jnp.int32).reshape(_NW, _NCHUNK, _CHUNK)
    gathered = _sc_gather(emb.reshape(_VW, _WIDE), idx)
    embeds = gathered.reshape(_B, _CTX * _EMB)
    b1r = b1.reshape(1, _HID)
    b2r = b2.reshape(1, _VOCAB)
    lse = _lse_call(embeds, W1, b1r, W2, b2r)
    return _out_call(embeds, W1, b1r, W2, b2r, lse)
